# baseline (speedup 1.0000x reference)
import functools

import jax
import jax.numpy as jnp
from jax import lax
from jax.experimental import pallas as pl

B, N, D = 32, 64, 256
NEG = -1000000000.0


def _decode_body(act_ref, wr_ref, br_ref, wq_ref, bq_ref, v_ref, vb_ref,
                 w1_ref, w2_ref, w3_ref, b123_ref, idx_ref, prob_ref):
    act2 = act_ref[...]
    a_t = jnp.dot(act2, wr_ref[...],
                  preferred_element_type=jnp.float32) + br_ref[...]
    a3 = a_t.reshape(N, B, D)
    act3 = act2.reshape(N, B, D)

    wq = wq_ref[...]
    bq = bq_ref[...]
    v_col = v_ref[...]
    vb = vb_ref[0, 0]
    w1 = w1_ref[...]
    w2 = w2_ref[...]
    w3 = w3_ref[...]
    b123 = b123_ref[...]

    iota_nb = lax.broadcasted_iota(jnp.int32, (N, B), 0)
    iota_nbd = lax.broadcasted_iota(jnp.int32, (N, B, D), 0)

    def score_and_pick(qs, mask_f):
        q = jnp.dot(qs, wq, preferred_element_type=jnp.float32) + bq
        th = jnp.tanh(a3 + q[None, :, :])
        sc = jnp.dot(th.reshape(N * B, D), v_col,
                     preferred_element_type=jnp.float32)
        scores = sc.reshape(N, B) + vb
        masked = jnp.where(mask_f > 0.5, NEG, scores)
        m = jnp.max(masked, axis=0, keepdims=True)
        idx = jnp.min(jnp.where(masked == m, iota_nb, N), axis=0,
                      keepdims=True)
        onehot = iota_nb == idx
        return masked, idx, onehot

    def gather_rows(idx):
        idx_b1 = jnp.transpose(idx)
        idx_bd = jnp.broadcast_to(idx_b1, (B, D))
        sel = iota_nbd == idx_bd[None, :, :]
        return jnp.sum(jnp.where(sel, act3, 0.0), axis=0)

    def body(t, carry):
        qs, a1, a2, mask_f, idx_acc = carry
        _, idx, onehot = score_and_pick(qs, mask_f)
        mask_f = jnp.maximum(mask_f, onehot.astype(jnp.float32))
        idx_acc = jnp.where(iota_nb == t, idx.astype(jnp.float32), idx_acc)
        next_action = gather_rows(idx)
        r1 = jnp.dot(next_action, w1, preferred_element_type=jnp.float32)
        r2 = jnp.dot(a1, w2, preferred_element_type=jnp.float32)
        r3 = jnp.dot(a2, w3, preferred_element_type=jnp.float32)
        qs = jnp.maximum(((r1 + r2) + r3) + b123, 0.0)
        return qs, next_action, a1, mask_f, idx_acc

    qs0 = act3[0, :, :]
    zeros_bd = qs0 * 0.0
    zeros_nb = iota_nb.astype(jnp.float32) * 0.0
    qs, a1, a2, mask_f, idx_acc = lax.fori_loop(
        0, N - 1, body, (qs0, zeros_bd, zeros_bd, zeros_nb, zeros_nb),
        unroll=21)

    masked, idx, onehot = score_and_pick(qs, mask_f)
    idx_acc = jnp.where(iota_nb == (N - 1), idx.astype(jnp.float32), idx_acc)
    m = jnp.max(masked, axis=0, keepdims=True)
    e = jnp.exp(masked - m)
    probs = e / jnp.sum(e, axis=0, keepdims=True)
    prob = lax.dot_general(probs, onehot.astype(jnp.float32),
                           (((0,), (0,)), ((), ())),
                           precision=lax.Precision.HIGHEST,
                           preferred_element_type=jnp.float32)
    idx_ref[...] = idx_acc.astype(jnp.int32)
    prob_ref[...] = prob


@functools.partial(jax.jit, static_argnames=())
def kernel(action_vectors, W_ref_k, W_ref_b, w_q_k, w_q_b, v_k, v_b,
           W1_k, W1_b, W2_k, W2_b, W3_k, W3_b):
    act2 = action_vectors.transpose(1, 0, 2).reshape(N * B, D)
    b123 = (W1_b + W2_b + W3_b).reshape(1, D)
    vb = v_b.reshape(1, 1)
    idx_nb, prob = pl.pallas_call(
        _decode_body,
        out_shape=(
            jax.ShapeDtypeStruct((N, B), jnp.int32),
            jax.ShapeDtypeStruct((B, B), jnp.float32),
        ),
    )(act2, W_ref_k, W_ref_b.reshape(1, D), w_q_k, w_q_b.reshape(1, D),
      v_k, vb, W1_k, W2_k, W3_k, b123)
    return idx_nb.T, prob


# speedup vs baseline: 1.0012x; 1.0012x over previous
import functools

import jax
import jax.numpy as jnp
from jax import lax
from jax.experimental import pallas as pl

B, N, D = 32, 64, 256
NEG = -1000000000.0


def _decode_body(act_ref, wr_ref, br_ref, wq_ref, bq_ref, v_ref, vb_ref,
                 w1_ref, w2_ref, w3_ref, b123_ref, idx_ref, prob_ref):
    act2 = act_ref[...]
    a_t = jnp.dot(act2, wr_ref[...],
                  preferred_element_type=jnp.float32) + br_ref[...]
    a3 = a_t.reshape(B, N, D)
    act3 = act2.reshape(B, N, D)
    act3_bf = act3.astype(jnp.bfloat16)

    wq = wq_ref[...]
    bq = bq_ref[...]
    v_col = v_ref[...]
    vb = vb_ref[0, 0]
    w1 = w1_ref[...]
    w2 = w2_ref[...]
    w3 = w3_ref[...]
    b123 = b123_ref[...]

    iota_n = lax.broadcasted_iota(jnp.int32, (B, N), 1)
    iota_n3_bf = lax.broadcasted_iota(jnp.int32, (B, N, D), 1).astype(
        jnp.bfloat16)

    def score_and_pick(qs, mask_f):
        q = jnp.dot(qs, wq, preferred_element_type=jnp.float32) + bq
        th = jnp.tanh(a3 + q[:, None, :])
        sc = jnp.dot(th.reshape(B * N, D), v_col,
                     preferred_element_type=jnp.float32)
        scores = sc.reshape(B, N) + vb
        masked = jnp.where(mask_f > 0.5, NEG, scores)
        m = jnp.max(masked, axis=-1, keepdims=True)
        idx = jnp.min(jnp.where(masked == m, iota_n, N), axis=-1,
                      keepdims=True)
        onehot = iota_n == idx
        return masked, idx, onehot

    def gather_rows(idx):
        idx_bd = jnp.broadcast_to(idx.astype(jnp.bfloat16), (B, D))
        sel = iota_n3_bf == idx_bd[:, None, :]
        picked = jnp.where(sel, act3_bf, jnp.bfloat16(0.0))
        return jnp.sum(picked, axis=1)

    def body(t, carry):
        qs, a1, a2, mask_f, idx_acc = carry
        _, idx, onehot = score_and_pick(qs, mask_f)
        mask_f = jnp.maximum(mask_f, onehot.astype(jnp.float32))
        idx_acc = jnp.where(iota_n == t, idx.astype(jnp.float32), idx_acc)
        next_action = gather_rows(idx)
        r1 = jnp.dot(next_action.astype(jnp.float32), w1,
                     preferred_element_type=jnp.float32)
        r2 = jnp.dot(a1.astype(jnp.float32), w2,
                     preferred_element_type=jnp.float32)
        r3 = jnp.dot(a2.astype(jnp.float32), w3,
                     preferred_element_type=jnp.float32)
        qs = jnp.maximum(((r1 + r2) + r3) + b123, 0.0)
        return qs, next_action, a1, mask_f, idx_acc

    qs0 = act3[:, 0, :]
    zeros_bd = act3_bf[:, 0, :] * jnp.bfloat16(0.0)
    zeros_bn = iota_n.astype(jnp.float32) * 0.0
    qs, a1, a2, mask_f, idx_acc = lax.fori_loop(
        0, N - 1, body, (qs0, zeros_bd, zeros_bd, zeros_bn, zeros_bn),
        unroll=21)

    masked, idx, onehot = score_and_pick(qs, mask_f)
    idx_acc = jnp.where(iota_n == (N - 1), idx.astype(jnp.float32), idx_acc)
    m = jnp.max(masked, axis=-1, keepdims=True)
    e = jnp.exp(masked - m)
    probs = e / jnp.sum(e, axis=-1, keepdims=True)
    prob = lax.dot_general(probs, onehot.astype(jnp.float32),
                           (((1,), (1,)), ((), ())),
                           precision=lax.Precision.HIGHEST,
                           preferred_element_type=jnp.float32)
    idx_ref[...] = idx_acc.astype(jnp.int32)
    prob_ref[...] = prob


@functools.partial(jax.jit, static_argnames=())
def kernel(action_vectors, W_ref_k, W_ref_b, w_q_k, w_q_b, v_k, v_b,
           W1_k, W1_b, W2_k, W2_b, W3_k, W3_b):
    act2 = action_vectors.reshape(B * N, D)
    b123 = (W1_b + W2_b + W3_b).reshape(1, D)
    vb = v_b.reshape(1, 1)
    idx, prob = pl.pallas_call(
        _decode_body,
        out_shape=(
            jax.ShapeDtypeStruct((B, N), jnp.int32),
            jax.ShapeDtypeStruct((B, B), jnp.float32),
        ),
    )(act2, W_ref_k, W_ref_b.reshape(1, D), w_q_k, w_q_b.reshape(1, D),
      v_k, vb, W1_k, W2_k, W3_k, b123)
    return idx, prob


# speedup vs baseline: 1.0483x; 1.0470x over previous
import functools

import jax
import jax.numpy as jnp
from jax import lax
from jax.experimental import pallas as pl

B, N, D = 32, 64, 256
NEG = -1000000000.0


def _decode_body(act_ref, actn_ref, wr_ref, br_ref, wq_ref, bq_ref, v_ref,
                 vb_ref, w1_ref, w2_ref, w3_ref, b123_ref, idx_ref, prob_ref):
    act2 = act_ref[...]
    a_t = jnp.dot(act2, wr_ref[...],
                  preferred_element_type=jnp.float32) + br_ref[...]
    a3 = a_t.reshape(B, N, D)
    actn3 = actn_ref[...].reshape(N, B, D)

    wq = wq_ref[...]
    bq = bq_ref[...]
    v_col = v_ref[...]
    vb = vb_ref[0, 0]
    w1 = w1_ref[...]
    w2 = w2_ref[...]
    w3 = w3_ref[...]
    b123 = b123_ref[...]

    iota_n = lax.broadcasted_iota(jnp.int32, (B, N), 1)
    iota_lead = lax.broadcasted_iota(jnp.int32, (N, B, D), 0)

    def score_and_pick(qs, mask_f):
        q = jnp.dot(qs, wq, preferred_element_type=jnp.float32) + bq
        th = jnp.tanh(a3 + q[:, None, :])
        sc = jnp.dot(th.reshape(B * N, D), v_col,
                     preferred_element_type=jnp.float32)
        scores = sc.reshape(B, N) + vb
        masked = jnp.where(mask_f > 0.5, NEG, scores)
        m = jnp.max(masked, axis=-1, keepdims=True)
        idx = jnp.min(jnp.where(masked == m, iota_n, N), axis=-1,
                      keepdims=True)
        onehot = iota_n == idx
        return masked, idx, onehot

    def gather_rows(idx):
        idx_bd = jnp.broadcast_to(idx, (B, D))
        sel = iota_lead == idx_bd[None, :, :]
        return jnp.sum(jnp.where(sel, actn3, 0.0), axis=0)

    def body(t, carry):
        qs, a1, a2, mask_f, idx_acc = carry
        _, idx, onehot = score_and_pick(qs, mask_f)
        mask_f = jnp.maximum(mask_f, onehot.astype(jnp.float32))
        idx_acc = jnp.where(iota_n == t, idx.astype(jnp.float32), idx_acc)
        next_action = gather_rows(idx)
        r1 = jnp.dot(next_action, w1, preferred_element_type=jnp.float32)
        r2 = jnp.dot(a1, w2, preferred_element_type=jnp.float32)
        r3 = jnp.dot(a2, w3, preferred_element_type=jnp.float32)
        qs = jnp.maximum(((r1 + r2) + r3) + b123, 0.0)
        return qs, next_action, a1, mask_f, idx_acc

    qs0 = actn3[0, :, :]
    zeros_bd = qs0 * 0.0
    zeros_bn = iota_n.astype(jnp.float32) * 0.0
    qs, a1, a2, mask_f, idx_acc = lax.fori_loop(
        0, N - 1, body, (qs0, zeros_bd, zeros_bd, zeros_bn, zeros_bn),
        unroll=21)

    masked, idx, onehot = score_and_pick(qs, mask_f)
    idx_acc = jnp.where(iota_n == (N - 1), idx.astype(jnp.float32), idx_acc)
    m = jnp.max(masked, axis=-1, keepdims=True)
    e = jnp.exp(masked - m)
    probs = e / jnp.sum(e, axis=-1, keepdims=True)
    prob = lax.dot_general(probs, onehot.astype(jnp.float32),
                           (((1,), (1,)), ((), ())),
                           precision=lax.Precision.HIGHEST,
                           preferred_element_type=jnp.float32)
    idx_ref[...] = idx_acc.astype(jnp.int32)
    prob_ref[...] = prob


@functools.partial(jax.jit, static_argnames=())
def kernel(action_vectors, W_ref_k, W_ref_b, w_q_k, w_q_b, v_k, v_b,
           W1_k, W1_b, W2_k, W2_b, W3_k, W3_b):
    act2 = action_vectors.reshape(B * N, D)
    actn2 = action_vectors.transpose(1, 0, 2).reshape(N * B, D)
    b123 = (W1_b + W2_b + W3_b).reshape(1, D)
    vb = v_b.reshape(1, 1)
    idx, prob = pl.pallas_call(
        _decode_body,
        out_shape=(
            jax.ShapeDtypeStruct((B, N), jnp.int32),
            jax.ShapeDtypeStruct((B, B), jnp.float32),
        ),
    )(act2, actn2, W_ref_k, W_ref_b.reshape(1, D), w_q_k,
      w_q_b.reshape(1, D), v_k, vb, W1_k, W2_k, W3_k, b123)
    return idx, prob


# speedup vs baseline: 1.1850x; 1.1304x over previous
import functools

import jax
import jax.numpy as jnp
from jax import lax
from jax.experimental import pallas as pl

B, N, D = 32, 64, 256
NEG = -1000000000.0


def _decode_body(act_ref, wr_ref, br_ref, wq_ref, bq_ref, v_ref, vb_ref,
                 w1_ref, w2_ref, w3_ref, b123_ref, idx_ref, prob_ref):
    act2 = act_ref[...]
    a_t = jnp.dot(act2, wr_ref[...],
                  preferred_element_type=jnp.float32) + br_ref[...]
    a3 = a_t.reshape(B, N, D)
    act3 = act2.reshape(B, N, D)

    wq = wq_ref[...]
    bq = bq_ref[...]
    v_col = v_ref[...]
    vb = vb_ref[0, 0]
    w1 = w1_ref[...]
    w2 = w2_ref[...]
    w3 = w3_ref[...]
    b123 = b123_ref[...]

    iota_n = lax.broadcasted_iota(jnp.int32, (B, N), 1)
    iota_n3 = lax.broadcasted_iota(jnp.int32, (B, N, D), 1)

    def score_and_pick(qs, mask_f):
        q = jnp.dot(qs, wq, preferred_element_type=jnp.float32) + bq
        th = jnp.tanh(a3 + q[:, None, :])
        sc = jnp.dot(th.reshape(B * N, D), v_col,
                     preferred_element_type=jnp.float32)
        scores = sc.reshape(B, N) + vb
        masked = jnp.where(mask_f > 0.5, NEG, scores)
        idx = jnp.argmax(masked, axis=-1, keepdims=True)
        onehot = iota_n == idx
        return masked, idx, onehot

    def gather_rows(idx):
        idx_bd = jnp.broadcast_to(idx, (B, D))
        sel = iota_n3 == idx_bd[:, None, :]
        return jnp.sum(jnp.where(sel, act3, 0.0), axis=1)

    def body(t, carry):
        qs, a1, a2, mask_f, idx_acc = carry
        _, idx, onehot = score_and_pick(qs, mask_f)
        mask_f = jnp.maximum(mask_f, onehot.astype(jnp.float32))
        idx_acc = jnp.where(iota_n == t, idx.astype(jnp.float32), idx_acc)
        next_action = gather_rows(idx)
        r1 = jnp.dot(next_action, w1, preferred_element_type=jnp.float32)
        r2 = jnp.dot(a1, w2, preferred_element_type=jnp.float32)
        r3 = jnp.dot(a2, w3, preferred_element_type=jnp.float32)
        qs = jnp.maximum(((r1 + r2) + r3) + b123, 0.0)
        return qs, next_action, a1, mask_f, idx_acc

    qs0 = act3[:, 0, :]
    zeros_bd = qs0 * 0.0
    zeros_bn = iota_n.astype(jnp.float32) * 0.0
    qs, a1, a2, mask_f, idx_acc = lax.fori_loop(
        0, N - 1, body, (qs0, zeros_bd, zeros_bd, zeros_bn, zeros_bn),
        unroll=21)

    masked, idx, onehot = score_and_pick(qs, mask_f)
    idx_acc = jnp.where(iota_n == (N - 1), idx.astype(jnp.float32), idx_acc)
    m = jnp.max(masked, axis=-1, keepdims=True)
    e = jnp.exp(masked - m)
    probs = e / jnp.sum(e, axis=-1, keepdims=True)
    prob = lax.dot_general(probs, onehot.astype(jnp.float32),
                           (((1,), (1,)), ((), ())),
                           precision=lax.Precision.HIGHEST,
                           preferred_element_type=jnp.float32)
    idx_ref[...] = idx_acc.astype(jnp.int32)
    prob_ref[...] = prob


@functools.partial(jax.jit, static_argnames=())
def kernel(action_vectors, W_ref_k, W_ref_b, w_q_k, w_q_b, v_k, v_b,
           W1_k, W1_b, W2_k, W2_b, W3_k, W3_b):
    act2 = action_vectors.reshape(B * N, D)
    b123 = (W1_b + W2_b + W3_b).reshape(1, D)
    vb = v_b.reshape(1, 1)
    idx, prob = pl.pallas_call(
        _decode_body,
        out_shape=(
            jax.ShapeDtypeStruct((B, N), jnp.int32),
            jax.ShapeDtypeStruct((B, B), jnp.float32),
        ),
    )(act2, W_ref_k, W_ref_b.reshape(1, D), w_q_k, w_q_b.reshape(1, D),
      v_k, vb, W1_k, W2_k, W3_k, b123)
    return idx, prob
